# baseline (speedup 1.0000x reference)
import functools

import jax
import jax.numpy as jnp
from jax import lax
from jax.experimental import pallas as pl
from jax.experimental.pallas import tpu as pltpu
from jax.experimental.pallas import tpu_sc as plsc

_NC = 2
_NS = 16
_NW = _NC * _NS
_SB = 8


def _rows_per_tile(n):
    return 128 * (-(-(n + 1) // (_NS * 128)))


def _make_deg_kernel(n, ch, k):
    rpt = _rows_per_tile(n)
    npad = rpt * _NS
    nz = rpt // k
    mesh = plsc.VectorSubcoreMesh(core_axis_name="c", subcore_axis_name="s")

    @functools.partial(
        pl.kernel, mesh=mesh,
        out_type=jax.ShapeDtypeStruct((_NC, npad, 16), jnp.float32),
        scratch_types=[
            pltpu.VMEM((ch, k), jnp.int32),
            pltpu.VMEM((k, 16), jnp.float32),
            pltpu.VMEM_SHARED((npad, 16), jnp.float32),
        ],
    )
    def deg_k(dst_hbm, out_hbm, dstv, onev, acc):
        c = lax.axis_index("c")
        s = lax.axis_index("s")
        w = c * _NS + s

        def zrow(i, carry):
            onev[i, :] = jnp.zeros((16,), jnp.float32)
            return carry

        lax.fori_loop(0, k, zrow, 0)
        for m in range(nz):
            pltpu.sync_copy(onev, acc.at[pl.ds(s * rpt + m * k, k), :])

        def orow(i, carry):
            onev[i, :] = jnp.ones((16,), jnp.float32)
            return carry

        lax.fori_loop(0, k, orow, 0)
        pltpu.sync_copy(dst_hbm.at[w], dstv)
        plsc.subcore_barrier()

        def chunk(j, carry):
            pltpu.sync_copy(onev, acc.at[dstv.at[j]], add=True)
            return carry

        lax.fori_loop(0, ch, chunk, 0)
        plsc.subcore_barrier()
        for m in range(nz):
            pltpu.sync_copy(acc.at[pl.ds(s * rpt + m * k, k), :], onev)
            pltpu.sync_copy(onev, out_hbm.at[c, pl.ds(s * rpt + m * k, k), :])

    return deg_k


def _make_agg_kernel(n, d, chi):
    L = 128
    sb = _SB
    nb = chi // sb
    rpt = _rows_per_tile(n)
    npad = rpt * _NS
    wchunks = []
    off = 0
    while off < rpt:
        sz = min(2 * L, rpt - off)
        wchunks.append((off, sz))
        off += sz
    mesh = plsc.VectorSubcoreMesh(core_axis_name="c", subcore_axis_name="s")

    @functools.partial(
        pl.kernel, mesh=mesh,
        out_type=jax.ShapeDtypeStruct((_NC, npad, d), jnp.float32),
        scratch_types=[
            pltpu.VMEM((2, sb, L), jnp.int32),
            pltpu.VMEM((2, sb, L), jnp.int32),
            pltpu.VMEM((2 * L, d), jnp.float32),
            pltpu.VMEM_SHARED((npad, d), jnp.float32),
            pltpu.SemaphoreType.DMA,
            pltpu.SemaphoreType.DMA,
            pltpu.SemaphoreType.DMA,
            pltpu.SemaphoreType.DMA,
            pltpu.SemaphoreType.DMA,
            pltpu.SemaphoreType.DMA,
        ],
    )
    def agg_k(g_hbm, src_hbm, dst_hbm, out_hbm, sidx, didx, rows, acc,
              si0, si1, sr0, sr1, ss0, ss1):
        c = lax.axis_index("c")
        s = lax.axis_index("s")
        w = c * _NS + s
        semi = [si0, si1]
        semr = [sr0, sr1]
        sems = [ss0, ss1]
        buf = [rows.at[pl.ds(0, L), :], rows.at[pl.ds(L, L), :]]

        def zrow(i, carry):
            for t in range(d // 16):
                rows[i, pl.ds(t * 16, 16)] = jnp.zeros((16,), jnp.float32)
            return carry

        lax.fori_loop(0, 2 * L, zrow, 0)
        for off, sz in wchunks:
            pltpu.sync_copy(rows.at[pl.ds(0, sz), :],
                            acc.at[pl.ds(s * rpt + off, sz), :])
        plsc.subcore_barrier()

        def load_idx(m, slot):
            pltpu.async_copy(src_hbm.at[w, pl.ds(m * sb, sb), :],
                             sidx.at[slot], semi[slot])
            pltpu.async_copy(dst_hbm.at[w, pl.ds(m * sb, sb), :],
                             didx.at[slot], semi[slot])

        def wait_idx(m, slot):
            pltpu.make_async_copy(src_hbm.at[w, pl.ds(m * sb, sb), :],
                                  sidx.at[slot], semi[slot]).wait()
            pltpu.make_async_copy(dst_hbm.at[w, pl.ds(m * sb, sb), :],
                                  didx.at[slot], semi[slot]).wait()

        load_idx(0, 0)

        def do_block(m, slot):
            wait_idx(m, slot)

            @pl.when(m + 1 < nb)
            def _():
                load_idx(m + 1, slot ^ 1)

            gd = [
                pltpu.async_copy(g_hbm.at[sidx.at[slot, 0]], buf[0], semr[0]),
                pltpu.async_copy(g_hbm.at[sidx.at[slot, 1]], buf[1], semr[1]),
            ]
            for j in range(sb):
                bb = j % 2
                gd[bb].wait()
                pltpu.sync_copy(buf[bb], acc.at[didx.at[slot, j]], add=True)
                if j + 2 < sb:
                    gd[bb] = pltpu.async_copy(
                        g_hbm.at[sidx.at[slot, j + 2]], buf[bb], semr[bb])

        def pair(mm, carry):
            for half in range(2):
                do_block(2 * mm + half, half)
            return carry

        lax.fori_loop(0, nb // 2, pair, 0)
        plsc.subcore_barrier()
        for off, sz in wchunks:
            pltpu.sync_copy(acc.at[pl.ds(s * rpt + off, sz), :],
                            rows.at[pl.ds(0, sz), :])
            pltpu.sync_copy(rows.at[pl.ds(0, sz), :],
                            out_hbm.at[c, pl.ds(s * rpt + off, sz), :])

    return agg_k


def _dinv_from(degp):
    deg = 1.0 + degp[0][:, 0:1] + degp[1][:, 0:1]
    return lax.rsqrt(deg)


def _mm_scale_body(x_ref, w_ref, degp_ref, o_ref):
    dinv = _dinv_from(degp_ref[...])
    o_ref[...] = jnp.dot(
        x_ref[...], w_ref[...], preferred_element_type=jnp.float32) * dinv


def _mid_body(p_ref, g1_ref, degp_ref, b1_ref, w2_ref, o_ref):
    dinv = _dinv_from(degp_ref[...])
    p = p_ref[...]
    h = (p[0] + p[1] + g1_ref[...]) * dinv + b1_ref[...]
    h = jnp.maximum(h, 0.0)
    o_ref[...] = jnp.dot(
        h, w2_ref[...], preferred_element_type=jnp.float32) * dinv


def _fin_body(q_ref, g2_ref, degp_ref, b2_ref, o_ref):
    dinv = _dinv_from(degp_ref[...])
    q = q_ref[...]
    o_ref[...] = (q[0] + q[1] + g2_ref[...]) * dinv + b2_ref[...]


def kernel(x, edge_index, W1, b1, W2, b2):
    n, d_in = x.shape
    d_hid = W1.shape[1]
    d_out = W2.shape[1]
    e = edge_index.shape[1]
    kd = 80
    chd = e // (kd * _NW)
    r = 400

    chi = 2 * _SB * (-(-e // (_NW * 128 * 2 * _SB)))
    e2 = _NW * chi * 128
    pad = e2 - e
    src_p = jnp.concatenate(
        [edge_index[0], jnp.zeros((pad,), edge_index.dtype)])
    dst_p = jnp.concatenate(
        [edge_index[1], jnp.full((pad,), n, edge_index.dtype)])

    dst3d_deg = edge_index[1].reshape(_NW, chd, kd)
    src3d = src_p.reshape(_NW, chi, 128)
    dst3d = dst_p.reshape(_NW, chi, 128)

    deg_k = _make_deg_kernel(n, chd, kd)
    agg_hid = _make_agg_kernel(n, d_hid, chi)

    degp = deg_k(dst3d_deg)

    full = lambda *shape: pl.BlockSpec(shape, lambda i: (0,) * len(shape))
    rows = lambda *shape: pl.BlockSpec((r,) + shape, lambda i: (i,) + (0,) * len(shape))
    degs = pl.BlockSpec((2, r, 16), lambda i: (0, i, 0))
    prt = lambda dd: pl.BlockSpec((2, r, dd), lambda i: (0, i, 0))

    g1 = pl.pallas_call(
        _mm_scale_body,
        grid=(n // r,),
        in_specs=[rows(d_in), full(d_in, d_hid), degs],
        out_specs=rows(d_hid),
        out_shape=jax.ShapeDtypeStruct((n, d_hid), jnp.float32),
    )(x, W1, degp)

    p = agg_hid(g1, src3d, dst3d)

    g2 = pl.pallas_call(
        _mid_body,
        grid=(n // r,),
        in_specs=[prt(d_hid), rows(d_hid), degs, full(1, d_hid), full(d_hid, d_out)],
        out_specs=rows(d_out),
        out_shape=jax.ShapeDtypeStruct((n, d_out), jnp.float32),
    )(p, g1, degp, b1.reshape(1, d_hid), W2)

    if d_out == d_hid:
        agg_out = agg_hid
    else:
        agg_out = _make_agg_kernel(n, d_out, chi)
    q = agg_out(g2, src3d, dst3d)

    out = pl.pallas_call(
        _fin_body,
        grid=(n // r,),
        in_specs=[prt(d_out), rows(d_out), degs, full(1, d_out)],
        out_specs=rows(d_out),
        out_shape=jax.ShapeDtypeStruct((n, d_out), jnp.float32),
    )(q, g2, degp, b2.reshape(1, d_out))

    return out


# speedup vs baseline: 3.0066x; 3.0066x over previous
import functools

import jax
import jax.numpy as jnp
from jax import lax
from jax.experimental import pallas as pl
from jax.experimental.pallas import tpu as pltpu
from jax.experimental.pallas import tpu_sc as plsc

_NC = 2
_NS = 16
_NW = _NC * _NS
_SB = 8


def _rows_per_tile(n):
    return 128 * (-(-(n + 1) // (_NS * 128)))


def _make_deg_kernel(n, ch, k):
    rpt = _rows_per_tile(n)
    npad = rpt * _NS
    nz = rpt // k
    mesh = plsc.VectorSubcoreMesh(core_axis_name="c", subcore_axis_name="s")

    @functools.partial(
        pl.kernel, mesh=mesh,
        out_type=jax.ShapeDtypeStruct((_NC, npad, 16), jnp.float32),
        scratch_types=[
            pltpu.VMEM((ch, k), jnp.int32),
            pltpu.VMEM((k, 16), jnp.float32),
            pltpu.VMEM_SHARED((npad, 16), jnp.float32),
        ],
    )
    def deg_k(dst_hbm, out_hbm, dstv, onev, acc):
        c = lax.axis_index("c")
        s = lax.axis_index("s")
        w = c * _NS + s

        def zrow(i, carry):
            onev[i, :] = jnp.zeros((16,), jnp.float32)
            return carry

        lax.fori_loop(0, k, zrow, 0)
        for m in range(nz):
            pltpu.sync_copy(onev, acc.at[pl.ds(s * rpt + m * k, k), :])

        def orow(i, carry):
            onev[i, :] = jnp.ones((16,), jnp.float32)
            return carry

        lax.fori_loop(0, k, orow, 0)
        pltpu.sync_copy(dst_hbm.at[w], dstv)
        plsc.subcore_barrier()

        def chunk(j, carry):
            pltpu.sync_copy(onev, acc.at[dstv.at[j]], add=True)
            return carry

        lax.fori_loop(0, ch, chunk, 0)
        plsc.subcore_barrier()
        for m in range(nz):
            pltpu.sync_copy(acc.at[pl.ds(s * rpt + m * k, k), :], onev)
            pltpu.sync_copy(onev, out_hbm.at[c, pl.ds(s * rpt + m * k, k), :])

    return deg_k


def _make_agg_kernel(n, d, chi):
    L = 128
    sb = _SB
    nb = chi // sb
    rpt = _rows_per_tile(n)
    npad = rpt * _NS
    wchunks = []
    off = 0
    while off < rpt:
        sz = min(2 * L, rpt - off)
        wchunks.append((off, sz))
        off += sz
    mesh = plsc.VectorSubcoreMesh(core_axis_name="c", subcore_axis_name="s")

    @functools.partial(
        pl.kernel, mesh=mesh,
        out_type=jax.ShapeDtypeStruct((_NC, npad, d), jnp.float32),
        scratch_types=[
            pltpu.VMEM((2, sb, L), jnp.int32),
            pltpu.VMEM((2, sb, L), jnp.int32),
            pltpu.VMEM((2 * L, d), jnp.float32),
            pltpu.VMEM_SHARED((npad, d), jnp.float32),
            pltpu.SemaphoreType.DMA,
            pltpu.SemaphoreType.DMA,
            pltpu.SemaphoreType.DMA,
            pltpu.SemaphoreType.DMA,
            pltpu.SemaphoreType.DMA,
            pltpu.SemaphoreType.DMA,
        ],
    )
    def agg_k(g_hbm, src_hbm, dst_hbm, out_hbm, sidx, didx, rows, acc,
              si0, si1, sr0, sr1, ss0, ss1):
        c = lax.axis_index("c")
        s = lax.axis_index("s")
        w = c * _NS + s
        semi = [si0, si1]
        semr = [sr0, sr1]
        sems = [ss0, ss1]
        buf = [rows.at[pl.ds(0, L), :], rows.at[pl.ds(L, L), :]]

        def zrow(i, carry):
            for t in range(d // 16):
                rows[i, pl.ds(t * 16, 16)] = jnp.zeros((16,), jnp.float32)
            return carry

        lax.fori_loop(0, 2 * L, zrow, 0)
        for off, sz in wchunks:
            pltpu.sync_copy(rows.at[pl.ds(0, sz), :],
                            acc.at[pl.ds(s * rpt + off, sz), :])
        plsc.subcore_barrier()

        def load_idx(m, slot):
            pltpu.async_copy(src_hbm.at[w, pl.ds(m * sb, sb), :],
                             sidx.at[slot], semi[slot])
            pltpu.async_copy(dst_hbm.at[w, pl.ds(m * sb, sb), :],
                             didx.at[slot], semi[slot])

        def wait_idx(m, slot):
            pltpu.make_async_copy(src_hbm.at[w, pl.ds(m * sb, sb), :],
                                  sidx.at[slot], semi[slot]).wait()
            pltpu.make_async_copy(dst_hbm.at[w, pl.ds(m * sb, sb), :],
                                  didx.at[slot], semi[slot]).wait()

        load_idx(0, 0)

        def do_block(m, slot):
            wait_idx(m, slot)

            @pl.when(m + 1 < nb)
            def _():
                load_idx(m + 1, slot ^ 1)

            gd = [
                pltpu.async_copy(g_hbm.at[sidx.at[slot, 0]], buf[0], semr[0]),
                pltpu.async_copy(g_hbm.at[sidx.at[slot, 1]], buf[1], semr[1]),
            ]
            for j in range(sb):
                bb = j % 2
                gd[bb].wait()
                pltpu.sync_copy(buf[bb], acc.at[didx.at[slot, j]], add=True)
                if j + 2 < sb:
                    gd[bb] = pltpu.async_copy(
                        g_hbm.at[sidx.at[slot, j + 2]], buf[bb], semr[bb])

        def pair(mm, carry):
            for half in range(2):
                do_block(2 * mm + half, half)
            return carry

        lax.fori_loop(0, nb // 2, pair, 0)
        plsc.subcore_barrier()
        for off, sz in wchunks:
            pltpu.sync_copy(acc.at[pl.ds(s * rpt + off, sz), :],
                            rows.at[pl.ds(0, sz), :])
            pltpu.sync_copy(rows.at[pl.ds(0, sz), :],
                            out_hbm.at[c, pl.ds(s * rpt + off, sz), :])

    return agg_k


def _dinv_from(degp):
    deg = 1.0 + degp[0][:, 0:1] + degp[1][:, 0:1]
    return lax.rsqrt(deg)


def _mm_scale_body(x_ref, w_ref, degp_ref, o_ref):
    dinv = _dinv_from(degp_ref[...])
    o_ref[...] = jnp.dot(
        x_ref[...], w_ref[...], preferred_element_type=jnp.float32) * dinv


def _mid_body(p_ref, g1_ref, degp_ref, b1_ref, w2_ref, o_ref):
    dinv = _dinv_from(degp_ref[...])
    p = p_ref[...]
    h = (p[0] + p[1] + g1_ref[...]) * dinv + b1_ref[...]
    h = jnp.maximum(h, 0.0)
    o_ref[...] = jnp.dot(
        h, w2_ref[...], preferred_element_type=jnp.float32) * dinv


def _fin_body(q_ref, g2_ref, degp_ref, b2_ref, o_ref):
    dinv = _dinv_from(degp_ref[...])
    q = q_ref[...]
    o_ref[...] = (q[0] + q[1] + g2_ref[...]) * dinv + b2_ref[...]


def kernel(x, edge_index, W1, b1, W2, b2):
    n, d_in = x.shape
    d_hid = W1.shape[1]
    d_out = W2.shape[1]
    e = edge_index.shape[1]
    kd = 80
    chd = e // (kd * _NW)
    r = 400

    chi = 2 * _SB * (-(-e // (_NW * 128 * 2 * _SB)))
    e2 = _NW * chi * 128
    pad = e2 - e
    rpt = _rows_per_tile(n)
    ngarb = rpt * _NS - n
    pad_i = jnp.arange(pad, dtype=edge_index.dtype)
    src_p = jnp.concatenate([edge_index[0], pad_i % n])
    dst_p = jnp.concatenate([edge_index[1], n + pad_i % ngarb])

    dst3d_deg = edge_index[1].reshape(_NW, chd, kd)
    src3d = src_p.reshape(_NW, chi, 128)
    dst3d = dst_p.reshape(_NW, chi, 128)

    deg_k = _make_deg_kernel(n, chd, kd)
    agg_hid = _make_agg_kernel(n, d_hid, chi)

    degp = deg_k(dst3d_deg)

    full = lambda *shape: pl.BlockSpec(shape, lambda i: (0,) * len(shape))
    rows = lambda *shape: pl.BlockSpec((r,) + shape, lambda i: (i,) + (0,) * len(shape))
    degs = pl.BlockSpec((2, r, 16), lambda i: (0, i, 0))
    prt = lambda dd: pl.BlockSpec((2, r, dd), lambda i: (0, i, 0))

    g1 = pl.pallas_call(
        _mm_scale_body,
        grid=(n // r,),
        in_specs=[rows(d_in), full(d_in, d_hid), degs],
        out_specs=rows(d_hid),
        out_shape=jax.ShapeDtypeStruct((n, d_hid), jnp.float32),
    )(x, W1, degp)

    p = agg_hid(g1, src3d, dst3d)

    g2 = pl.pallas_call(
        _mid_body,
        grid=(n // r,),
        in_specs=[prt(d_hid), rows(d_hid), degs, full(1, d_hid), full(d_hid, d_out)],
        out_specs=rows(d_out),
        out_shape=jax.ShapeDtypeStruct((n, d_out), jnp.float32),
    )(p, g1, degp, b1.reshape(1, d_hid), W2)

    if d_out == d_hid:
        agg_out = agg_hid
    else:
        agg_out = _make_agg_kernel(n, d_out, chi)
    q = agg_out(g2, src3d, dst3d)

    out = pl.pallas_call(
        _fin_body,
        grid=(n // r,),
        in_specs=[prt(d_out), rows(d_out), degs, full(1, d_out)],
        out_specs=rows(d_out),
        out_shape=jax.ShapeDtypeStruct((n, d_out), jnp.float32),
    )(q, g2, degp, b2.reshape(1, d_out))

    return out


# speedup vs baseline: 3.0096x; 1.0010x over previous
import functools

import jax
import jax.numpy as jnp
from jax import lax
from jax.experimental import pallas as pl
from jax.experimental.pallas import tpu as pltpu
from jax.experimental.pallas import tpu_sc as plsc

_NC = 2
_NS = 16
_NW = _NC * _NS
_SB = 8


def _rows_per_tile(n):
    return 128 * (-(-(n + 1) // (_NS * 128)))


def _make_deg_kernel(n, ch, k):
    rpt = _rows_per_tile(n)
    npad = rpt * _NS
    nz = rpt // k
    mesh = plsc.VectorSubcoreMesh(core_axis_name="c", subcore_axis_name="s")

    @functools.partial(
        pl.kernel, mesh=mesh,
        out_type=jax.ShapeDtypeStruct((_NC, npad, 16), jnp.float32),
        scratch_types=[
            pltpu.VMEM((ch, k), jnp.int32),
            pltpu.VMEM((k, 16), jnp.float32),
            pltpu.VMEM_SHARED((npad, 16), jnp.float32),
        ],
    )
    def deg_k(dst_hbm, out_hbm, dstv, onev, acc):
        c = lax.axis_index("c")
        s = lax.axis_index("s")
        w = c * _NS + s

        def zrow(i, carry):
            onev[i, :] = jnp.zeros((16,), jnp.float32)
            return carry

        lax.fori_loop(0, k, zrow, 0)
        for m in range(nz):
            pltpu.sync_copy(onev, acc.at[pl.ds(s * rpt + m * k, k), :])

        def orow(i, carry):
            onev[i, :] = jnp.ones((16,), jnp.float32)
            return carry

        lax.fori_loop(0, k, orow, 0)
        pltpu.sync_copy(dst_hbm.at[w], dstv)
        plsc.subcore_barrier()

        def chunk(j, carry):
            pltpu.sync_copy(onev, acc.at[dstv.at[j]], add=True)
            return carry

        lax.fori_loop(0, ch, chunk, 0)
        plsc.subcore_barrier()
        pltpu.sync_copy(acc.at[pl.ds(s * rpt, rpt), :],
                        out_hbm.at[c, pl.ds(s * rpt, rpt), :])

    return deg_k


def _make_agg_kernel(n, d, chi):
    L = 128
    sb = _SB
    nb = chi // sb
    rpt = _rows_per_tile(n)
    npad = rpt * _NS
    wchunks = []
    off = 0
    while off < rpt:
        sz = min(2 * L, rpt - off)
        wchunks.append((off, sz))
        off += sz
    mesh = plsc.VectorSubcoreMesh(core_axis_name="c", subcore_axis_name="s")

    @functools.partial(
        pl.kernel, mesh=mesh,
        out_type=jax.ShapeDtypeStruct((_NC, npad, d), jnp.float32),
        scratch_types=[
            pltpu.VMEM((2, sb, L), jnp.int32),
            pltpu.VMEM((2, sb, L), jnp.int32),
            pltpu.VMEM((2 * L, d), jnp.float32),
            pltpu.VMEM_SHARED((npad, d), jnp.float32),
            pltpu.SemaphoreType.DMA,
            pltpu.SemaphoreType.DMA,
            pltpu.SemaphoreType.DMA,
            pltpu.SemaphoreType.DMA,
            pltpu.SemaphoreType.DMA,
            pltpu.SemaphoreType.DMA,
        ],
    )
    def agg_k(g_hbm, src_hbm, dst_hbm, out_hbm, sidx, didx, rows, acc,
              si0, si1, sr0, sr1, ss0, ss1):
        c = lax.axis_index("c")
        s = lax.axis_index("s")
        w = c * _NS + s
        semi = [si0, si1]
        semr = [sr0, sr1]
        sems = [ss0, ss1]
        buf = [rows.at[pl.ds(0, L), :], rows.at[pl.ds(L, L), :]]

        def zrow(i, carry):
            for t in range(d // 16):
                rows[i, pl.ds(t * 16, 16)] = jnp.zeros((16,), jnp.float32)
            return carry

        lax.fori_loop(0, 2 * L, zrow, 0)
        for off, sz in wchunks:
            pltpu.sync_copy(rows.at[pl.ds(0, sz), :],
                            acc.at[pl.ds(s * rpt + off, sz), :])
        plsc.subcore_barrier()

        def load_idx(m, slot):
            pltpu.async_copy(src_hbm.at[w, pl.ds(m * sb, sb), :],
                             sidx.at[slot], semi[slot])
            pltpu.async_copy(dst_hbm.at[w, pl.ds(m * sb, sb), :],
                             didx.at[slot], semi[slot])

        def wait_idx(m, slot):
            pltpu.make_async_copy(src_hbm.at[w, pl.ds(m * sb, sb), :],
                                  sidx.at[slot], semi[slot]).wait()
            pltpu.make_async_copy(dst_hbm.at[w, pl.ds(m * sb, sb), :],
                                  didx.at[slot], semi[slot]).wait()

        load_idx(0, 0)

        def do_block(m, slot):
            wait_idx(m, slot)

            @pl.when(m + 1 < nb)
            def _():
                load_idx(m + 1, slot ^ 1)

            gd = [
                pltpu.async_copy(g_hbm.at[sidx.at[slot, 0]], buf[0], semr[0]),
                pltpu.async_copy(g_hbm.at[sidx.at[slot, 1]], buf[1], semr[1]),
            ]
            for j in range(sb):
                bb = j % 2
                gd[bb].wait()
                pltpu.sync_copy(buf[bb], acc.at[didx.at[slot, j]], add=True)
                if j + 2 < sb:
                    gd[bb] = pltpu.async_copy(
                        g_hbm.at[sidx.at[slot, j + 2]], buf[bb], semr[bb])

        def pair(mm, carry):
            for half in range(2):
                do_block(2 * mm + half, half)
            return carry

        lax.fori_loop(0, nb // 2, pair, 0)
        plsc.subcore_barrier()
        pltpu.sync_copy(acc.at[pl.ds(s * rpt, rpt), :],
                        out_hbm.at[c, pl.ds(s * rpt, rpt), :])

    return agg_k


def _dinv_from(degp):
    deg = 1.0 + degp[0][:, 0:1] + degp[1][:, 0:1]
    return lax.rsqrt(deg)


def _mm_scale_body(x_ref, w_ref, degp_ref, o_ref):
    dinv = _dinv_from(degp_ref[...])
    o_ref[...] = jnp.dot(
        x_ref[...], w_ref[...], preferred_element_type=jnp.float32) * dinv


def _mid_body(p_ref, g1_ref, degp_ref, b1_ref, w2_ref, o_ref):
    dinv = _dinv_from(degp_ref[...])
    p = p_ref[...]
    h = (p[0] + p[1] + g1_ref[...]) * dinv + b1_ref[...]
    h = jnp.maximum(h, 0.0)
    o_ref[...] = jnp.dot(
        h, w2_ref[...], preferred_element_type=jnp.float32) * dinv


def _fin_body(q_ref, g2_ref, degp_ref, b2_ref, o_ref):
    dinv = _dinv_from(degp_ref[...])
    q = q_ref[...]
    o_ref[...] = (q[0] + q[1] + g2_ref[...]) * dinv + b2_ref[...]


def kernel(x, edge_index, W1, b1, W2, b2):
    n, d_in = x.shape
    d_hid = W1.shape[1]
    d_out = W2.shape[1]
    e = edge_index.shape[1]
    kd = 80
    chd = e // (kd * _NW)
    r = 400

    chi = 2 * _SB * (-(-e // (_NW * 128 * 2 * _SB)))
    e2 = _NW * chi * 128
    pad = e2 - e
    rpt = _rows_per_tile(n)
    ngarb = rpt * _NS - n
    pad_i = jnp.arange(pad, dtype=edge_index.dtype)
    src_p = jnp.concatenate([edge_index[0], pad_i % n])
    dst_p = jnp.concatenate([edge_index[1], n + pad_i % ngarb])

    dst3d_deg = edge_index[1].reshape(_NW, chd, kd)
    src3d = src_p.reshape(_NW, chi, 128)
    dst3d = dst_p.reshape(_NW, chi, 128)

    deg_k = _make_deg_kernel(n, chd, kd)
    agg_hid = _make_agg_kernel(n, d_hid, chi)

    degp = deg_k(dst3d_deg)

    full = lambda *shape: pl.BlockSpec(shape, lambda i: (0,) * len(shape))
    rows = lambda *shape: pl.BlockSpec((r,) + shape, lambda i: (i,) + (0,) * len(shape))
    degs = pl.BlockSpec((2, r, 16), lambda i: (0, i, 0))
    prt = lambda dd: pl.BlockSpec((2, r, dd), lambda i: (0, i, 0))

    g1 = pl.pallas_call(
        _mm_scale_body,
        grid=(n // r,),
        in_specs=[rows(d_in), full(d_in, d_hid), degs],
        out_specs=rows(d_hid),
        out_shape=jax.ShapeDtypeStruct((n, d_hid), jnp.float32),
    )(x, W1, degp)

    p = agg_hid(g1, src3d, dst3d)

    g2 = pl.pallas_call(
        _mid_body,
        grid=(n // r,),
        in_specs=[prt(d_hid), rows(d_hid), degs, full(1, d_hid), full(d_hid, d_out)],
        out_specs=rows(d_out),
        out_shape=jax.ShapeDtypeStruct((n, d_out), jnp.float32),
    )(p, g1, degp, b1.reshape(1, d_hid), W2)

    if d_out == d_hid:
        agg_out = agg_hid
    else:
        agg_out = _make_agg_kernel(n, d_out, chi)
    q = agg_out(g2, src3d, dst3d)

    out = pl.pallas_call(
        _fin_body,
        grid=(n // r,),
        in_specs=[prt(d_out), rows(d_out), degs, full(1, d_out)],
        out_specs=rows(d_out),
        out_shape=jax.ShapeDtypeStruct((n, d_out), jnp.float32),
    )(q, g2, degp, b2.reshape(1, d_out))

    return out


# speedup vs baseline: 3.0115x; 1.0006x over previous
import functools

import jax
import jax.numpy as jnp
from jax import lax
from jax.experimental import pallas as pl
from jax.experimental.pallas import tpu as pltpu
from jax.experimental.pallas import tpu_sc as plsc

_NC = 2
_NS = 16
_NW = _NC * _NS
_SB = 8


def _rows_per_tile(n):
    return 128 * (-(-(n + 1) // (_NS * 128)))


def _make_deg_kernel(n, ch, k):
    rpt = _rows_per_tile(n)
    npad = rpt * _NS
    nz = rpt // k
    mesh = plsc.VectorSubcoreMesh(core_axis_name="c", subcore_axis_name="s")

    @functools.partial(
        pl.kernel, mesh=mesh,
        out_type=jax.ShapeDtypeStruct((_NC, npad, 16), jnp.float32),
        scratch_types=[
            pltpu.VMEM((ch, k), jnp.int32),
            pltpu.VMEM((k, 16), jnp.float32),
            pltpu.VMEM_SHARED((npad, 16), jnp.float32),
        ],
    )
    def deg_k(dst_hbm, out_hbm, dstv, onev, acc):
        c = lax.axis_index("c")
        s = lax.axis_index("s")
        w = c * _NS + s

        def zrow(i, carry):
            onev[i, :] = jnp.zeros((16,), jnp.float32)
            return carry

        lax.fori_loop(0, k, zrow, 0)
        for m in range(nz):
            pltpu.sync_copy(onev, acc.at[pl.ds(s * rpt + m * k, k), :])

        def orow(i, carry):
            onev[i, :] = jnp.ones((16,), jnp.float32)
            return carry

        lax.fori_loop(0, k, orow, 0)
        pltpu.sync_copy(dst_hbm.at[w], dstv)
        plsc.subcore_barrier()

        def chunk(j, carry):
            pltpu.sync_copy(onev, acc.at[dstv.at[j]], add=True)
            return carry

        lax.fori_loop(0, ch, chunk, 0)
        plsc.subcore_barrier()
        pltpu.sync_copy(acc.at[pl.ds(s * rpt, rpt), :],
                        out_hbm.at[c, pl.ds(s * rpt, rpt), :])

    return deg_k


def _make_agg_kernel(n, d, chi):
    L = 128
    sb = _SB
    nb = chi // sb
    rpt = _rows_per_tile(n)
    npad = rpt * _NS
    wchunks = []
    off = 0
    while off < rpt:
        sz = min(2 * L, rpt - off)
        wchunks.append((off, sz))
        off += sz
    mesh = plsc.VectorSubcoreMesh(core_axis_name="c", subcore_axis_name="s")

    @functools.partial(
        pl.kernel, mesh=mesh,
        out_type=jax.ShapeDtypeStruct((_NC, npad, d), jnp.float32),
        scratch_types=[
            pltpu.VMEM((2, sb, L), jnp.int32),
            pltpu.VMEM((2, sb, L), jnp.int32),
            pltpu.VMEM((2 * L, d), jnp.float32),
            pltpu.VMEM_SHARED((npad, d), jnp.float32),
            pltpu.SemaphoreType.DMA,
            pltpu.SemaphoreType.DMA,
            pltpu.SemaphoreType.DMA,
            pltpu.SemaphoreType.DMA,
            pltpu.SemaphoreType.DMA,
            pltpu.SemaphoreType.DMA,
        ],
    )
    def agg_k(g_hbm, src_hbm, dst_hbm, out_hbm, sidx, didx, rows, acc,
              si0, si1, sr0, sr1, ss0, ss1):
        c = lax.axis_index("c")
        s = lax.axis_index("s")
        w = c * _NS + s
        semi = [si0, si1]
        semr = [sr0, sr1]
        sems = [ss0, ss1]
        buf = [rows.at[pl.ds(0, L), :], rows.at[pl.ds(L, L), :]]

        def zrow(i, carry):
            for t in range(d // 16):
                rows[i, pl.ds(t * 16, 16)] = jnp.zeros((16,), jnp.float32)
            return carry

        lax.fori_loop(0, 2 * L, zrow, 0)
        for off, sz in wchunks:
            pltpu.sync_copy(rows.at[pl.ds(0, sz), :],
                            acc.at[pl.ds(s * rpt + off, sz), :])
        plsc.subcore_barrier()

        def load_idx(m, slot):
            pltpu.async_copy(src_hbm.at[w, pl.ds(m * sb, sb), :],
                             sidx.at[slot], semi[slot])
            pltpu.async_copy(dst_hbm.at[w, pl.ds(m * sb, sb), :],
                             didx.at[slot], semi[slot])

        def wait_idx(m, slot):
            pltpu.make_async_copy(src_hbm.at[w, pl.ds(m * sb, sb), :],
                                  sidx.at[slot], semi[slot]).wait()
            pltpu.make_async_copy(dst_hbm.at[w, pl.ds(m * sb, sb), :],
                                  didx.at[slot], semi[slot]).wait()

        load_idx(0, 0)

        def do_block(m, slot):
            wait_idx(m, slot)

            @pl.when(m + 1 < nb)
            def _():
                load_idx(m + 1, slot ^ 1)

            gd = [
                pltpu.async_copy(g_hbm.at[sidx.at[slot, 0]], buf[0], semr[0]),
                pltpu.async_copy(g_hbm.at[sidx.at[slot, 1]], buf[1], semr[1]),
            ]
            for j in range(sb):
                bb = j % 2
                gd[bb].wait()
                pltpu.sync_copy(buf[bb], acc.at[didx.at[slot, j]], add=True)
                if j + 2 < sb:
                    gd[bb] = pltpu.async_copy(
                        g_hbm.at[sidx.at[slot, j + 2]], buf[bb], semr[bb])

        def pair(mm, carry):
            for half in range(2):
                do_block(2 * mm + half, half)
            return carry

        lax.fori_loop(0, nb // 2, pair, 0)
        plsc.subcore_barrier()
        pltpu.sync_copy(acc.at[pl.ds(s * rpt, rpt), :],
                        out_hbm.at[c, pl.ds(s * rpt, rpt), :])

    return agg_k


def _dinv_from(degp):
    deg = 1.0 + degp[0][:, 0:1] + degp[1][:, 0:1]
    return lax.rsqrt(deg)


def _mm_body(x_ref, w_ref, o_ref):
    o_ref[...] = jnp.dot(
        x_ref[...], w_ref[...], preferred_element_type=jnp.float32)


def _scale_body(h_ref, degp_ref, o_ref):
    dinv = _dinv_from(degp_ref[...])
    o_ref[...] = h_ref[...] * dinv


def _mid_body(p_ref, g1_ref, degp_ref, b1_ref, w2_ref, o_ref):
    dinv = _dinv_from(degp_ref[...])
    p = p_ref[...]
    h = (p[0] + p[1] + g1_ref[...]) * dinv + b1_ref[...]
    h = jnp.maximum(h, 0.0)
    o_ref[...] = jnp.dot(
        h, w2_ref[...], preferred_element_type=jnp.float32) * dinv


def _fin_body(q_ref, g2_ref, degp_ref, b2_ref, o_ref):
    dinv = _dinv_from(degp_ref[...])
    q = q_ref[...]
    o_ref[...] = (q[0] + q[1] + g2_ref[...]) * dinv + b2_ref[...]


def kernel(x, edge_index, W1, b1, W2, b2):
    n, d_in = x.shape
    d_hid = W1.shape[1]
    d_out = W2.shape[1]
    e = edge_index.shape[1]
    kd = 80
    chd = e // (kd * _NW)
    r = 400

    chi = 2 * _SB * (-(-e // (_NW * 128 * 2 * _SB)))
    e2 = _NW * chi * 128
    pad = e2 - e
    rpt = _rows_per_tile(n)
    ngarb = rpt * _NS - n
    pad_i = jnp.arange(pad, dtype=edge_index.dtype)
    src_p = jnp.concatenate([edge_index[0], pad_i % n])
    dst_p = jnp.concatenate([edge_index[1], n + pad_i % ngarb])

    dst3d_deg = edge_index[1].reshape(_NW, chd, kd)
    src3d = src_p.reshape(_NW, chi, 128)
    dst3d = dst_p.reshape(_NW, chi, 128)

    deg_k = _make_deg_kernel(n, chd, kd)
    agg_hid = _make_agg_kernel(n, d_hid, chi)

    degp = deg_k(dst3d_deg)

    full = lambda *shape: pl.BlockSpec(shape, lambda i: (0,) * len(shape))
    rows = lambda *shape: pl.BlockSpec((r,) + shape, lambda i: (i,) + (0,) * len(shape))
    degs = pl.BlockSpec((2, r, 16), lambda i: (0, i, 0))
    prt = lambda dd: pl.BlockSpec((2, r, dd), lambda i: (0, i, 0))

    h1 = pl.pallas_call(
        _mm_body,
        grid=(n // r,),
        in_specs=[rows(d_in), full(d_in, d_hid)],
        out_specs=rows(d_hid),
        out_shape=jax.ShapeDtypeStruct((n, d_hid), jnp.float32),
    )(x, W1)

    g1 = pl.pallas_call(
        _scale_body,
        grid=(n // r,),
        in_specs=[rows(d_hid), degs],
        out_specs=rows(d_hid),
        out_shape=jax.ShapeDtypeStruct((n, d_hid), jnp.float32),
    )(h1, degp)

    p = agg_hid(g1, src3d, dst3d)

    g2 = pl.pallas_call(
        _mid_body,
        grid=(n // r,),
        in_specs=[prt(d_hid), rows(d_hid), degs, full(1, d_hid), full(d_hid, d_out)],
        out_specs=rows(d_out),
        out_shape=jax.ShapeDtypeStruct((n, d_out), jnp.float32),
    )(p, g1, degp, b1.reshape(1, d_hid), W2)

    if d_out == d_hid:
        agg_out = agg_hid
    else:
        agg_out = _make_agg_kernel(n, d_out, chi)
    q = agg_out(g2, src3d, dst3d)

    out = pl.pallas_call(
        _fin_body,
        grid=(n // r,),
        in_specs=[prt(d_out), rows(d_out), degs, full(1, d_out)],
        out_specs=rows(d_out),
        out_shape=jax.ShapeDtypeStruct((n, d_out), jnp.float32),
    )(q, g2, degp, b2.reshape(1, d_out))

    return out
